# baseline (speedup 1.0000x reference)
import functools

import jax
import jax.numpy as jnp
from jax import lax
from jax.experimental import pallas as pl
from jax.experimental.pallas import tpu as pltpu
from jax.experimental.pallas import tpu_sc as plsc

_NC = 2
_NS = 16
_NW = _NC * _NS
_LANES = 16
_TILE_R = 8
_DEPTH = 8


@functools.partial(jax.jit, static_argnames=("n_rows", "d", "seq_len"))
def _sc_embed(table, idx2d, pos, *, n_rows, d, seq_len):
    b_per_w = n_rows // _NW
    rows_per_w = b_per_w // 128
    n_groups = b_per_w // _LANES
    h = d // 2
    nch = h // _LANES

    mesh = plsc.VectorSubcoreMesh(
        core_axis_name="c", subcore_axis_name="s",
        num_cores=_NC, num_subcores=_NS)

    scratch = [
        pltpu.VMEM((rows_per_w, 128), jnp.int32),
        pltpu.VMEM((seq_len, d), jnp.float32),
    ]
    scratch += [pltpu.VMEM((1, d), jnp.float32)] * (_DEPTH * _LANES)
    scratch += [pltpu.VMEM((_LANES, h), jnp.float32)] * (2 * _DEPTH)
    scratch += [pltpu.SemaphoreType.DMA] * (2 * _DEPTH)

    @functools.partial(
        pl.kernel,
        out_type=(jax.ShapeDtypeStruct((n_rows, h), jnp.float32),
                  jax.ShapeDtypeStruct((n_rows, h), jnp.float32)),
        mesh=mesh,
        scratch_types=scratch,
        compiler_params=pltpu.CompilerParams(
            use_tc_tiling_on_sc=True, needs_layout_passes=False),
    )
    def k(table_hbm, idx_hbm, pos_hbm, re_hbm, im_hbm,
          idx_v, pos_v, *bufs_sems):
        bank = tuple(bufs_sems[i * _LANES:(i + 1) * _LANES]
                     for i in range(_DEPTH))
        st = bufs_sems[_DEPTH * _LANES:_DEPTH * _LANES + 2 * _DEPTH]
        stage = tuple((st[2 * i], st[2 * i + 1]) for i in range(_DEPTH))
        rest = bufs_sems[_DEPTH * _LANES + 2 * _DEPTH:]
        sems = rest[:_DEPTH]
        sem_w = rest[_DEPTH:2 * _DEPTH]
        wid = lax.axis_index("s") * _NC + lax.axis_index("c")
        base = wid * b_per_w
        pltpu.sync_copy(idx_hbm.at[pl.ds(wid * rows_per_w, rows_per_w), :],
                        idx_v)
        pltpu.sync_copy(pos_hbm, pos_v)

        lanes_iota = lax.iota(jnp.int32, _LANES)
        int_min = jnp.int32(-2**31)

        def group_vec(g):
            return idx_v[g >> 3, pl.ds((g & 7) * _LANES, _LANES)]

        def extract(g):
            vec = group_vec(g)
            return tuple(
                lax.reduce_max(
                    jnp.where(lanes_iota == lane, vec, int_min), axes=(0,))
                for lane in range(_LANES))

        def fire(ts, b):
            for lane in range(_LANES):
                t = ts[lane]
                pltpu.async_copy(
                    table_hbm.at[t >> 3,
                                 pl.ds(jnp.bitwise_and(t, _TILE_R - 1), 1), :],
                    bank[b][lane], sems[b])

        def drain(b):
            for lane in range(_LANES):
                pltpu.make_async_copy(table_hbm.at[0, pl.ds(0, 1), :],
                                      bank[b][lane], sems[b]).wait()

        def process(ts, g, b, p):
            for lane in range(_LANES):
                lp = jnp.bitwise_and(g * _LANES + lane, seq_len - 1)
                buf = bank[b][lane]
                for c in range(nch):
                    s = pl.ds(c * _LANES, _LANES)
                    s2 = pl.ds(h + c * _LANES, _LANES)
                    stage[p][0][lane, s] = buf[0, s] + pos_v[lp, s]
                    stage[p][1][lane, s] = buf[0, s2] + pos_v[lp, s2]

        def stage_out(g, p):
            dst = pl.ds(base + g * _LANES, _LANES)
            pltpu.async_copy(stage[p][0], re_hbm.at[dst, :], sem_w[p])
            pltpu.async_copy(stage[p][1], im_hbm.at[dst, :], sem_w[p])

        def stage_drain(p):
            for sref in (stage[p][0], stage[p][1]):
                pltpu.make_async_copy(
                    sref, re_hbm.at[pl.ds(0, _LANES), :], sem_w[p]).wait()

        n_body = n_groups // _DEPTH
        ts_init = []
        for i in range(_DEPTH):
            tsi = extract(i)
            fire(tsi, i)
            ts_init.append(tsi)

        def one(g, b, ts_cur, m):
            drain(b)

            @pl.when(m >= 1)
            def _():
                stage_drain(b)

            process(ts_cur, g, b, b)
            stage_out(g, b)
            ts_n = extract(jnp.minimum(g + _DEPTH, n_groups - 1))

            @pl.when(g + _DEPTH <= n_groups - 1)
            def _():
                fire(ts_n, b)

            return ts_n

        def body(m, carry):
            g0 = m * _DEPTH
            return tuple(one(g0 + i, i, carry[i], m) for i in range(_DEPTH))

        lax.fori_loop(0, n_body, body, tuple(ts_init))
        for i in range(_DEPTH):
            stage_drain(i)

    return k(table, idx2d, pos)


def kernel(x, token_table, pos_embedding):
    B, L = x.shape
    d = token_table.shape[1]
    n_rows = B * L
    idx2d = x.reshape(n_rows // 128, 128).astype(jnp.int32)
    pos = pos_embedding[0, :L, :]
    table3 = token_table.reshape(-1, _TILE_R, d)
    re, im = _sc_embed(table3, idx2d, pos, n_rows=n_rows, d=d, seq_len=L)
    re = re.reshape(B, L, d // 2)
    im = im.reshape(B, L, d // 2)
    return jax.lax.complex(re, im)


# speedup vs baseline: 1.0504x; 1.0504x over previous
import functools

import jax
import jax.numpy as jnp
from jax import lax
from jax.experimental import pallas as pl
from jax.experimental.pallas import tpu as pltpu
from jax.experimental.pallas import tpu_sc as plsc

_NC = 2
_NS = 16
_NW = _NC * _NS
_LANES = 16
_TILE_R = 8
_DEPTH = 4


@functools.partial(jax.jit, static_argnames=("n_rows", "d", "seq_len"))
def _sc_embed(table, idx2d, pos, *, n_rows, d, seq_len):
    b_per_w = n_rows // _NW
    rows_per_w = b_per_w // 128
    n_groups = b_per_w // _LANES
    h = d // 2
    nch = h // _LANES

    mesh = plsc.VectorSubcoreMesh(
        core_axis_name="c", subcore_axis_name="s",
        num_cores=_NC, num_subcores=_NS)

    scratch = [
        pltpu.VMEM((rows_per_w, 128), jnp.int32),
        pltpu.VMEM((seq_len, d), jnp.float32),
    ]
    scratch += [pltpu.VMEM((1, d), jnp.float32)] * (_DEPTH * _LANES)
    scratch += [pltpu.VMEM((_LANES, h), jnp.float32)] * (2 * _DEPTH)
    scratch += [pltpu.SemaphoreType.DMA] * (2 * _DEPTH)

    @functools.partial(
        pl.kernel,
        out_type=(jax.ShapeDtypeStruct((n_rows, h), jnp.float32),
                  jax.ShapeDtypeStruct((n_rows, h), jnp.float32)),
        mesh=mesh,
        scratch_types=scratch,
        compiler_params=pltpu.CompilerParams(
            use_tc_tiling_on_sc=True, needs_layout_passes=False),
    )
    def k(table_hbm, idx_hbm, pos_hbm, re_hbm, im_hbm,
          idx_v, pos_v, *bufs_sems):
        bank = tuple(bufs_sems[i * _LANES:(i + 1) * _LANES]
                     for i in range(_DEPTH))
        st = bufs_sems[_DEPTH * _LANES:_DEPTH * _LANES + 2 * _DEPTH]
        stage = tuple((st[2 * i], st[2 * i + 1]) for i in range(_DEPTH))
        rest = bufs_sems[_DEPTH * _LANES + 2 * _DEPTH:]
        sems = rest[:_DEPTH]
        sem_w = rest[_DEPTH:2 * _DEPTH]
        wid = lax.axis_index("s") * _NC + lax.axis_index("c")
        base = wid * b_per_w
        pltpu.sync_copy(idx_hbm.at[pl.ds(wid * rows_per_w, rows_per_w), :],
                        idx_v)
        pltpu.sync_copy(pos_hbm, pos_v)

        lanes_iota = lax.iota(jnp.int32, _LANES)
        int_min = jnp.int32(-2**31)

        def group_vec(g):
            return idx_v[g >> 3, pl.ds((g & 7) * _LANES, _LANES)]

        def extract(g):
            vec = group_vec(g)
            return tuple(
                lax.reduce_max(
                    jnp.where(lanes_iota == lane, vec, int_min), axes=(0,))
                for lane in range(_LANES))

        def fire(ts, b):
            for lane in range(_LANES):
                t = ts[lane]
                pltpu.async_copy(
                    table_hbm.at[t >> 3,
                                 pl.ds(jnp.bitwise_and(t, _TILE_R - 1), 1), :],
                    bank[b][lane], sems[b])

        def drain(b):
            for lane in range(_LANES):
                pltpu.make_async_copy(table_hbm.at[0, pl.ds(0, 1), :],
                                      bank[b][lane], sems[b]).wait()

        def process(ts, g, b, p):
            for lane in range(_LANES):
                lp = jnp.bitwise_and(g * _LANES + lane, seq_len - 1)
                buf = bank[b][lane]
                for c in range(nch):
                    s = pl.ds(c * _LANES, _LANES)
                    s2 = pl.ds(h + c * _LANES, _LANES)
                    stage[p][0][lane, s] = buf[0, s] + pos_v[lp, s]
                    stage[p][1][lane, s] = buf[0, s2] + pos_v[lp, s2]

        def stage_out(g, p):
            dst = pl.ds(base + g * _LANES, _LANES)
            pltpu.async_copy(stage[p][0], re_hbm.at[dst, :], sem_w[p])
            pltpu.async_copy(stage[p][1], im_hbm.at[dst, :], sem_w[p])

        def stage_drain(p):
            for sref in (stage[p][0], stage[p][1]):
                pltpu.make_async_copy(
                    sref, re_hbm.at[pl.ds(0, _LANES), :], sem_w[p]).wait()

        n_body = n_groups // _DEPTH
        ts_init = []
        for i in range(_DEPTH):
            tsi = extract(i)
            fire(tsi, i)
            ts_init.append(tsi)

        def one(g, b, ts_cur, m):
            drain(b)

            @pl.when(m >= 1)
            def _():
                stage_drain(b)

            process(ts_cur, g, b, b)
            stage_out(g, b)
            ts_n = extract(jnp.minimum(g + _DEPTH, n_groups - 1))

            @pl.when(g + _DEPTH <= n_groups - 1)
            def _():
                fire(ts_n, b)

            return ts_n

        def body(m, carry):
            g0 = m * _DEPTH
            return tuple(one(g0 + i, i, carry[i], m) for i in range(_DEPTH))

        lax.fori_loop(0, n_body, body, tuple(ts_init))
        for i in range(_DEPTH):
            stage_drain(i)

    return k(table, idx2d, pos)


def kernel(x, token_table, pos_embedding):
    B, L = x.shape
    d = token_table.shape[1]
    n_rows = B * L
    idx2d = x.reshape(n_rows // 128, 128).astype(jnp.int32)
    pos = pos_embedding[0, :L, :]
    table3 = token_table.reshape(-1, _TILE_R, d)
    re, im = _sc_embed(table3, idx2d, pos, n_rows=n_rows, d=d, seq_len=L)
    re = re.reshape(B, L, d // 2)
    im = im.reshape(B, L, d // 2)
    return jax.lax.complex(re, im)
